# baseline (speedup 1.0000x reference)
import functools

import jax
import jax.numpy as jnp
from jax import lax
from jax.experimental import pallas as pl
from jax.experimental.pallas import tpu as pltpu
from jax.experimental.pallas import tpu_sc as plsc

_F = 26
_K = 16
_CARD = 100000


def _build_fm_call(B):
    info = plsc.get_sparse_core_info()
    NC, NS = info.num_cores, info.num_subcores
    NW = NC * NS
    bw = B // NW
    CB = 64
    NCH = bw // CB
    CI = CB * _F
    GW = 128
    NG = CI // GW
    assert B % NW == 0 and bw % CB == 0 and CI % GW == 0 and NCH % 2 == 0

    mesh = plsc.VectorSubcoreMesh(core_axis_name="c", subcore_axis_name="s")

    @functools.partial(
        pl.kernel,
        mesh=mesh,
        compiler_params=pltpu.CompilerParams(use_tc_tiling_on_sc=False),
        out_type=jax.ShapeDtypeStruct((B,), jnp.float32),
        scratch_types=[
            pltpu.VMEM((CI,), jnp.int32),
            pltpu.VMEM((CI,), jnp.int32),
            pltpu.VMEM((CI, _K), jnp.float32),
            pltpu.VMEM((CI, _K), jnp.float32),
            pltpu.VMEM((CI + 16,), jnp.float32),
            pltpu.VMEM((CI + 16,), jnp.float32),
            pltpu.VMEM((CI,), jnp.int32),
            pltpu.VMEM((CB,), jnp.float32),
            pltpu.SemaphoreType.DMA,
            pltpu.SemaphoreType.DMA,
        ],
    )
    def fm(x_hbm, offs_hbm, emb_hbm, lin_hbm, out_hbm,
           idx0, idx1, rows0, rows1, linv0, linv1, offs_v, out_v,
           sem0, sem1):
        wid = lax.axis_index("s") * NC + lax.axis_index("c")
        base = wid * bw

        pltpu.sync_copy(offs_hbm, offs_v)
        zero16 = jnp.zeros((16,), jnp.float32)
        linv0[pl.ds(CI, 16)] = zero16
        linv1[pl.ds(CI, 16)] = zero16
        lane = lax.iota(jnp.int32, 16)
        tail_mask = jnp.where(lane < (_F - 16), 1.0, 0.0)

        dnums = lax.GatherDimensionNumbers(
            offset_dims=(), collapsed_slice_dims=(0,), start_index_map=(0,))
        rot_idx = [((lane + sh) & 15)[:, None] for sh in (8, 4, 2, 1)]

        def lane_sum(v):
            for ri in rot_idx:
                v = v + lax.gather(
                    v, ri, dnums, (1,),
                    mode=lax.GatherScatterMode.PROMISE_IN_BOUNDS)
            return v

        idx = (idx0, idx1)
        rows = (rows0, rows1)
        linv = (linv0, linv1)
        sems = (sem0, sem1)

        def stage(c, s):
            start = (base + c * CB) * _F
            pltpu.sync_copy(x_hbm.at[pl.ds(start, CI)], idx[s])

            def add_offs(i, carry):
                sl = pl.ds(pl.multiple_of(i * 16, 16), 16)
                idx[s][sl] = idx[s][sl] + offs_v[sl]
                return carry

            lax.fori_loop(0, CI // 16, add_offs, 0)
            for g in range(NG):
                gs = pl.ds(g * GW, GW)
                pltpu.make_async_copy(
                    emb_hbm.at[idx[s].at[gs]], rows[s].at[gs, :], sems[s]
                ).start()
                pltpu.make_async_copy(
                    lin_hbm.at[idx[s].at[gs]], linv[s].at[gs], sems[s]
                ).start()

        def drain(s):
            for g in range(NG):
                gs = pl.ds(g * GW, GW)
                pltpu.make_async_copy(
                    emb_hbm.at[idx[s].at[gs]], rows[s].at[gs, :], sems[s]
                ).wait()
                pltpu.make_async_copy(
                    lin_hbm.at[idx[s].at[gs]], linv[s].at[gs], sems[s]
                ).wait()

        def compute(c, s):
            r_ref = rows[s]
            l_ref = linv[s]

            def grp(g, carry):
                res = zero16
                for j in range(16):
                    roff = (g * 16 + j) * _F
                    r = r_ref[roff]
                    acc_s = r
                    acc_q = r * r
                    for f in range(1, _F):
                        r = r_ref[roff + f]
                        acc_s = acc_s + r
                        acc_q = acc_q + r * r
                    t = acc_s * acc_s - acc_q
                    v0 = l_ref[pl.ds(roff, 16)]
                    v1 = l_ref[pl.ds(roff + 16, 16)]
                    val = lane_sum(0.5 * t + v0 + tail_mask * v1)
                    res = jnp.where(lane == j, val, res)
                sl = pl.ds(pl.multiple_of(g * 16, 16), 16)
                out_v[sl] = res
                return carry

            lax.fori_loop(0, CB // 16, grp, 0)
            pltpu.sync_copy(out_v, out_hbm.at[pl.ds(base + c * CB, CB)])

        stage(0, 0)

        def pipe(i, carry):
            c = i * 2
            stage(c + 1, 1)
            drain(0)
            compute(c, 0)

            @pl.when(c + 2 < NCH)
            def _():
                stage(c + 2, 0)

            drain(1)
            compute(c + 1, 1)
            return carry

        lax.fori_loop(0, NCH // 2, pipe, 0)

    return fm


def kernel(x, emb_table, linear_table, bias):
    B, F = x.shape
    x_flat = x.reshape(-1).astype(jnp.int32)
    offsets = jnp.arange(F, dtype=jnp.int32) * _CARD
    offs_pat = jnp.tile(offsets, 64)
    out = _build_fm_call(B)(x_flat, offs_pat, emb_table, linear_table)
    return out.reshape(B, 1) + bias[None, :]
